# baseline (speedup 1.0000x reference)
import jax
import jax.numpy as jnp
from jax import lax
from jax.experimental import pallas as pl
from jax.experimental.pallas import tpu as pltpu
from jax.experimental.pallas import tpu_sc as plsc

NUM_TOKENS = 32768
NUM_EXPERTS = 64
TOP_K = 8
LANES = 16
NUM_CORES = 2
NUM_SUBCORES = 16
NUM_WORKERS = NUM_CORES * NUM_SUBCORES
TOK_PER_W = NUM_TOKENS // NUM_WORKERS
ET = NUM_EXPERTS // TOP_K
TT = NUM_TOKENS // 128
TT_PER_W = TOK_PER_W // 128
UNROLL = 8
TPAD = 137


def _merge_top8(a, ai, b, bi, lo_mask):
    rb = jnp.flip(b)
    rbi = jnp.flip(bi)
    d = jnp.where(lo_mask, a, rb)
    di = jnp.where(lo_mask, ai, rbi)
    return plsc.sort_key_val(d, di, descending=True)


def _tec_body(logits_hbm, w_hbm, id_hbm, id2_hbm, logits_v, w_v, id_v):
    wid = lax.axis_index("s") * NUM_CORES + lax.axis_index("c")
    for et in range(ET):
        pltpu.sync_copy(logits_hbm.at[et, pl.ds(wid * TT_PER_W, TT_PER_W)],
                        logits_v.at[et, :, :, pl.ds(0, 128)])

    iota = lax.iota(jnp.int32, LANES)
    lo_mask = iota < TOP_K
    et_lo = iota >> 3
    e_vec = iota & 7

    def token_top8(t):
        tl = t // 128
        tloc = t % 128
        f_tl = jnp.full((LANES,), 0, jnp.int32) + tl
        f_tloc = jnp.full((LANES,), 0, jnp.int32) + tloc
        sorted_chunks = []
        for q in range(NUM_EXPERTS // LANES):
            v = plsc.load_gather(
                logits_v, [et_lo + 2 * q, f_tl, e_vec, f_tloc])
            sorted_chunks.append(
                plsc.sort_key_val(v, iota + q * LANES, descending=True))
        (a, ai), (b, bi), (c, ci), (d, di) = sorted_chunks
        m0, m0i = _merge_top8(a, ai, b, bi, lo_mask)
        m1, m1i = _merge_top8(c, ci, d, di, lo_mask)
        top, topi = _merge_top8(m0, m0i, m1, m1i, lo_mask)
        plsc.store_scatter(w_v, [f_tl, iota, f_tloc], top, mask=lo_mask)
        plsc.store_scatter(id_v, [f_tl, iota, f_tloc], topi, mask=lo_mask)

    @plsc.parallel_loop(0, TOK_PER_W, unroll=UNROLL)
    def pass1(t):
        token_top8(t)

    @plsc.parallel_loop(0, TOK_PER_W // LANES, unroll=4)
    def pass2(g):
        tl = g // 8
        t0 = (g % 8) * LANES
        f_tl = jnp.full((LANES,), 0, jnp.int32) + tl
        t_vec = t0 + iota
        cols = [jnp.full((LANES,), r, jnp.int32) for r in range(TOP_K)]
        vals = [plsc.load_gather(w_v, [f_tl, cols[r], t_vec])
                for r in range(TOP_K)]
        es = [jnp.exp(v - vals[0]) for v in vals]
        s = es[0]
        for e in es[1:]:
            s = s + e
        for r in range(TOP_K):
            plsc.store_scatter(w_v, [f_tl, cols[r], t_vec], es[r] / s)

    pltpu.sync_copy(w_v.at[:, :, pl.ds(0, 128)],
                    w_hbm.at[pl.ds(wid * TT_PER_W, TT_PER_W)])
    pltpu.sync_copy(id_v.at[:, :, pl.ds(0, 128)],
                    id_hbm.at[pl.ds(wid * TT_PER_W, TT_PER_W)])
    pltpu.sync_copy(id_v.at[:, :, pl.ds(0, 128)],
                    id2_hbm.at[pl.ds(wid * TT_PER_W, TT_PER_W)])


@jax.jit
def _select_topk(router_logits_fp32):
    mesh = plsc.VectorSubcoreMesh(
        core_axis_name="c", subcore_axis_name="s",
        num_cores=NUM_CORES, num_subcores=NUM_SUBCORES)
    fn = pl.kernel(
        _tec_body,
        out_type=(
            jax.ShapeDtypeStruct((TT, TOP_K, 128), jnp.float32),
            jax.ShapeDtypeStruct((TT, TOP_K, 128), jnp.int32),
            jax.ShapeDtypeStruct((TT, TOP_K, 128), jnp.int32),
        ),
        mesh=mesh,
        compiler_params=pltpu.CompilerParams(
            needs_layout_passes=False, use_tc_tiling_on_sc=False),
        scratch_types=[
            pltpu.VMEM((ET, TT_PER_W, TOP_K, TPAD), jnp.float32),
            pltpu.VMEM((TT_PER_W, TOP_K, TPAD), jnp.float32),
            pltpu.VMEM((TT_PER_W, TOP_K, TPAD), jnp.int32),
        ],
    )
    a = router_logits_fp32.reshape(TT, 128, ET, TOP_K).transpose(2, 0, 3, 1)
    o_w, o_id, o_id2 = fn(a)
    weights = o_w.transpose(0, 2, 1).reshape(NUM_TOKENS, TOP_K)
    ids = o_id.transpose(0, 2, 1).reshape(NUM_TOKENS, TOP_K)
    ids2 = o_id2.transpose(0, 2, 1).reshape(NUM_TOKENS, TOP_K)
    return weights, ids, ids2


def kernel(router_logits_fp32, topk_ids, topk_weights):
    del topk_ids, topk_weights
    weights, ids, ids2 = _select_topk(router_logits_fp32)
    return (weights, ids, ids2)


# speedup vs baseline: 1.1151x; 1.1151x over previous
import jax
import jax.numpy as jnp
from jax import lax
from jax.experimental import pallas as pl
from jax.experimental.pallas import tpu as pltpu
from jax.experimental.pallas import tpu_sc as plsc

NUM_TOKENS = 32768
NUM_EXPERTS = 64
TOP_K = 8
LANES = 16
NUM_CORES = 2
NUM_SUBCORES = 16
NUM_WORKERS = NUM_CORES * NUM_SUBCORES
TOK_PER_W = NUM_TOKENS // NUM_WORKERS
ET = NUM_EXPERTS // TOP_K
TT = NUM_TOKENS // 128
TT_PER_W = TOK_PER_W // 128
UNROLL = 8
TPAD = 144


def _merge_top8(a, ai, b, bi, lo_mask):
    rb = jnp.flip(b)
    rbi = jnp.flip(bi)
    d = jnp.where(lo_mask, a, rb)
    di = jnp.where(lo_mask, ai, rbi)
    return plsc.sort_key_val(d, di, descending=True)


def _tec_body(logits_hbm, w_hbm, id_hbm, id2_hbm, logits_v, w_v, id_v,
              dma_sem):
    wid = lax.axis_index("s") * NUM_CORES + lax.axis_index("c")
    copies = [
        pltpu.async_copy(logits_hbm.at[et, pl.ds(wid * TT_PER_W, TT_PER_W)],
                         logits_v.at[et, :, :, pl.ds(0, 128)], dma_sem)
        for et in range(ET)]
    for c in copies:
        c.wait()

    iota = lax.iota(jnp.int32, LANES)
    lo_mask = iota < TOP_K
    et_lo = iota >> 3
    e_vec = iota & 7

    def token_top8(t):
        tl = t // 128
        tloc = t % 128
        f_tl = jnp.full((LANES,), 0, jnp.int32) + tl
        f_tloc = jnp.full((LANES,), 0, jnp.int32) + tloc
        sorted_chunks = []
        for q in range(NUM_EXPERTS // LANES):
            v = plsc.load_gather(
                logits_v, [et_lo + 2 * q, f_tl, e_vec, f_tloc])
            sorted_chunks.append(
                plsc.sort_key_val(v, iota + q * LANES, descending=True))
        (a, ai), (b, bi), (c, ci), (d, di) = sorted_chunks
        m0, m0i = _merge_top8(a, ai, b, bi, lo_mask)
        m1, m1i = _merge_top8(c, ci, d, di, lo_mask)
        top, topi = _merge_top8(m0, m0i, m1, m1i, lo_mask)
        plsc.store_scatter(w_v, [f_tl, iota, f_tloc], top, mask=lo_mask)
        plsc.store_scatter(id_v, [f_tl, iota, f_tloc], topi, mask=lo_mask)

    @plsc.parallel_loop(0, TOK_PER_W, unroll=UNROLL)
    def pass1(t):
        token_top8(t)

    @plsc.parallel_loop(0, TOK_PER_W // LANES, unroll=4)
    def pass2(g):
        tl = g // 8
        t0 = (g % 8) * LANES
        f_tl = jnp.full((LANES,), 0, jnp.int32) + tl
        t_vec = t0 + iota
        cols = [jnp.full((LANES,), r, jnp.int32) for r in range(TOP_K)]
        vals = [plsc.load_gather(w_v, [f_tl, cols[r], t_vec])
                for r in range(TOP_K)]
        es = [jnp.exp(v - vals[0]) for v in vals]
        s = es[0]
        for e in es[1:]:
            s = s + e
        for r in range(TOP_K):
            plsc.store_scatter(w_v, [f_tl, cols[r], t_vec], es[r] / s)

    out_copies = [
        pltpu.async_copy(w_v.at[:, :, pl.ds(0, 128)],
                         w_hbm.at[pl.ds(wid * TT_PER_W, TT_PER_W)], dma_sem),
        pltpu.async_copy(id_v.at[:, :, pl.ds(0, 128)],
                         id_hbm.at[pl.ds(wid * TT_PER_W, TT_PER_W)], dma_sem),
        pltpu.async_copy(id_v.at[:, :, pl.ds(0, 128)],
                         id2_hbm.at[pl.ds(wid * TT_PER_W, TT_PER_W)], dma_sem),
    ]
    for c in out_copies:
        c.wait()


@jax.jit
def _select_topk(router_logits_fp32):
    mesh = plsc.VectorSubcoreMesh(
        core_axis_name="c", subcore_axis_name="s",
        num_cores=NUM_CORES, num_subcores=NUM_SUBCORES)
    fn = pl.kernel(
        _tec_body,
        out_type=(
            jax.ShapeDtypeStruct((TT, TOP_K, 128), jnp.float32),
            jax.ShapeDtypeStruct((TT, TOP_K, 128), jnp.int32),
            jax.ShapeDtypeStruct((TT, TOP_K, 128), jnp.int32),
        ),
        mesh=mesh,
        compiler_params=pltpu.CompilerParams(
            needs_layout_passes=False, use_tc_tiling_on_sc=False),
        scratch_types=[
            pltpu.VMEM((ET, TT_PER_W, TOP_K, TPAD), jnp.float32),
            pltpu.VMEM((TT_PER_W, TOP_K, TPAD), jnp.float32),
            pltpu.VMEM((TT_PER_W, TOP_K, TPAD), jnp.int32),
            pltpu.SemaphoreType.DMA,
        ],
    )
    a = router_logits_fp32.reshape(TT, 128, ET, TOP_K).transpose(2, 0, 3, 1)
    o_w, o_id, o_id2 = fn(a)
    weights = o_w.transpose(0, 2, 1).reshape(NUM_TOKENS, TOP_K)
    ids = o_id.transpose(0, 2, 1).reshape(NUM_TOKENS, TOP_K)
    ids2 = o_id2.transpose(0, 2, 1).reshape(NUM_TOKENS, TOP_K)
    return weights, ids, ids2


def kernel(router_logits_fp32, topk_ids, topk_weights):
    del topk_ids, topk_weights
    weights, ids, ids2 = _select_topk(router_logits_fp32)
    return (weights, ids, ids2)
